# baseline (speedup 1.0000x reference)
import functools

import jax
import jax.numpy as jnp
import numpy as np
from jax import lax
from jax.experimental import pallas as pl
from jax.experimental.pallas import tpu as pltpu

_VOXEL_SIZE = (0.1, 0.1, 0.2)
_PC_RANGE = (-54.0, -54.0, -5.0, 54.0, 54.0, 3.0)
_MAX_PTS = 10
_MAX_VOX = 120000
_GRID = (1080, 1080, 40)
_LANES = 128


def _ceil_to(x, m):
    return (x + m - 1) // m * m


def _swar_popcount(x):
    x = x - ((x >> 1) & 0x55555555)
    x = (x & 0x33333333) + ((x >> 2) & 0x33333333)
    x = (x + (x >> 4)) & 0x0F0F0F0F
    return (x * 0x01010101) >> 24


def _kernel_body(n_pts, n_cells, max_vox, max_pts, grid, vsz, rmin,
                 nrows_bm,
                 f0, f1, f2, f3, f4,
                 vox_ref, coord_ref, cnt_ref,
                 bm_ref, wp_ref, lin_ref, pack_ref):
    gx, gy, gz = grid
    lane1 = lax.broadcasted_iota(jnp.int32, (1, _LANES), 1)
    flat2 = (lax.broadcasted_iota(jnp.int32, (2, _LANES), 0) * _LANES
             + lax.broadcasted_iota(jnp.int32, (2, _LANES), 1))

    x = f0[...]
    y = f1[...]
    z = f2[...]
    cx = jnp.floor((x - rmin[0]) / vsz[0]).astype(jnp.int32)
    cy = jnp.floor((y - rmin[1]) / vsz[1]).astype(jnp.int32)
    cz = jnp.floor((z - rmin[2]) / vsz[2]).astype(jnp.int32)
    nrows_p = f0.shape[0]
    flat_idx = (lax.broadcasted_iota(jnp.int32, (nrows_p, _LANES), 0) * _LANES
                + lax.broadcasted_iota(jnp.int32, (nrows_p, _LANES), 1))
    valid = ((cx >= 0) & (cx < gx) & (cy >= 0) & (cy < gy)
             & (cz >= 0) & (cz < gz) & (flat_idx < n_pts))
    lin = (cz * gy + cy) * gx + cx
    lin_ref[...] = jnp.where(valid, lin, n_cells)
    pack_ref[...] = jnp.where(valid, (cz << 22) | (cy << 11) | cx, 0)

    vox_ref[...] = jnp.zeros_like(vox_ref)
    coord_ref[...] = jnp.zeros_like(coord_ref)
    cnt_ref[...] = jnp.zeros_like(cnt_ref)
    bm_ref[...] = jnp.zeros_like(bm_ref)

    n_chunks = (n_pts + 15) // 16

    def bm_body(cidx, c):
        ri = cidx >> 3
        sh = (_LANES - ((cidx & 7) << 4)) & (_LANES - 1)
        lrow = pltpu.roll(lin_ref[pl.ds(ri, 1), :], sh, axis=1)
        for k in range(16):
            l = lrow[0, k]
            w = l >> 5
            bit = jnp.int32(1) << (l & 31)
            r = w >> 7
            q = w & 127
            row = bm_ref[pl.ds(r, 1), :]
            bm_ref[pl.ds(r, 1), :] = row | jnp.where(lane1 == q, bit, 0)
        return c

    lax.fori_loop(0, n_chunks, bm_body, 0)

    pc = _swar_popcount(bm_ref[...]).astype(jnp.float32)
    i_k = lax.broadcasted_iota(jnp.int32, (_LANES, _LANES), 0)
    i_j = lax.broadcasted_iota(jnp.int32, (_LANES, _LANES), 1)
    m_strict = (i_k < i_j).astype(jnp.float32)
    intra = lax.dot_general(pc, m_strict, (((1,), (0,)), ((), ())),
                            preferred_element_type=jnp.float32)
    wp_ref[...] = intra

    def carry_body(r, c):
        cf = c.astype(jnp.float32)
        wrow = wp_ref[pl.ds(r, 1), :]
        brow = bm_ref[pl.ds(r, 1), :]
        row_excl_last = wrow[0, 127]
        last_pc = _swar_popcount(brow[0, 127])
        wp_ref[pl.ds(r, 1), :] = wrow + cf
        return c + row_excl_last.astype(jnp.int32) + last_pc

    lax.fori_loop(0, nrows_bm, carry_body, jnp.int32(0))

    def scatter_body(cidx, c):
        ri = cidx >> 3
        sh = (_LANES - ((cidx & 7) << 4)) & (_LANES - 1)
        lrow = pltpu.roll(lin_ref[pl.ds(ri, 1), :], sh, axis=1)
        prow = pltpu.roll(pack_ref[pl.ds(ri, 1), :], sh, axis=1)
        f0r = pltpu.roll(f0[pl.ds(ri, 1), :], sh, axis=1)
        f1r = pltpu.roll(f1[pl.ds(ri, 1), :], sh, axis=1)
        f2r = pltpu.roll(f2[pl.ds(ri, 1), :], sh, axis=1)
        f3r = pltpu.roll(f3[pl.ds(ri, 1), :], sh, axis=1)
        f4r = pltpu.roll(f4[pl.ds(ri, 1), :], sh, axis=1)

        for k in range(16):
            l = lrow[0, k]

            @pl.when(l < n_cells)
            def _(l=l, k=k):
                w = l >> 5
                b = l & 31
                wr = w >> 7
                wsh = (_LANES - (w & 127)) & (_LANES - 1)
                word = pltpu.roll(bm_ref[pl.ds(wr, 1), :], wsh, axis=1)[0, 0]
                wpf = pltpu.roll(wp_ref[pl.ds(wr, 1), :], wsh, axis=1)[0, 0]
                below = _swar_popcount(word & ((jnp.int32(1) << b) - 1))
                v = wpf.astype(jnp.int32) + below

                @pl.when(v < max_vox)
                def _():
                    vr = v >> 7
                    vq = v & 127
                    crow = cnt_ref[pl.ds(vr, 1), :]
                    slot = pltpu.roll(
                        crow, (_LANES - vq) & (_LANES - 1), axis=1)[0, 0]

                    @pl.when(slot < max_pts)
                    def _():
                        base = (v * max_pts + slot) * 5
                        br = base >> 7
                        bq = base & 127
                        blk = vox_ref[pl.ds(br, 2), :]
                        blk = jnp.where(flat2 == bq, f0r[0, k], blk)
                        blk = jnp.where(flat2 == bq + 1, f1r[0, k], blk)
                        blk = jnp.where(flat2 == bq + 2, f2r[0, k], blk)
                        blk = jnp.where(flat2 == bq + 3, f3r[0, k], blk)
                        blk = jnp.where(flat2 == bq + 4, f4r[0, k], blk)
                        vox_ref[pl.ds(br, 2), :] = blk

                        p = prow[0, k]
                        cb = v * 3
                        cr = cb >> 7
                        cq = cb & 127
                        cblk = coord_ref[pl.ds(cr, 2), :]
                        cblk = jnp.where(flat2 == cq, p >> 22, cblk)
                        cblk = jnp.where(flat2 == cq + 1,
                                         (p >> 11) & 0x7FF, cblk)
                        cblk = jnp.where(flat2 == cq + 2, p & 0x7FF, cblk)
                        coord_ref[pl.ds(cr, 2), :] = cblk

                        cnt_ref[pl.ds(vr, 1), :] = jnp.where(
                            lane1 == vq, slot + 1, crow)

        return c

    lax.fori_loop(0, (n_pts + 15) // 16, scatter_body, 0)


def _voxelize(points, grid, vsz, rmin, max_vox, max_pts, interpret=False):
    n_pts = points.shape[0]
    gx, gy, gz = grid
    n_cells = gx * gy * gz
    n_words = _ceil_to(n_cells // 32 + 2, _LANES)
    nrows_bm = n_words // _LANES
    npad = _ceil_to(n_pts, _LANES)
    nrows_p = npad // _LANES

    cols = []
    for k in range(5):
        c = jnp.pad(points[:, k], (0, npad - n_pts))
        cols.append(c.reshape(nrows_p, _LANES))

    nr_vox = _ceil_to(max_vox * max_pts * 5, _LANES) // _LANES + 1
    nr_coord = _ceil_to(max_vox * 3, _LANES) // _LANES + 1
    nr_cnt = _ceil_to(max_vox, _LANES) // _LANES + 1

    body = functools.partial(
        _kernel_body, n_pts, n_cells, max_vox, max_pts, grid, vsz, rmin,
        nrows_bm)
    vox, coord, cnt = pl.pallas_call(
        body,
        out_shape=(
            jax.ShapeDtypeStruct((nr_vox, _LANES), jnp.float32),
            jax.ShapeDtypeStruct((nr_coord, _LANES), jnp.int32),
            jax.ShapeDtypeStruct((nr_cnt, _LANES), jnp.int32),
        ),
        scratch_shapes=[
            pltpu.VMEM((nrows_bm, _LANES), jnp.int32),
            pltpu.VMEM((nrows_bm, _LANES), jnp.float32),
            pltpu.VMEM((nrows_p, _LANES), jnp.int32),
            pltpu.VMEM((nrows_p, _LANES), jnp.int32),
        ],
        interpret=interpret,
    )(*cols)

    voxels = vox.reshape(-1)[: max_vox * max_pts * 5].reshape(
        max_vox, max_pts, 5)
    coordinates = coord.reshape(-1)[: max_vox * 3].reshape(max_vox, 3)
    num_points = cnt.reshape(-1)[: max_vox]
    return voxels, coordinates, num_points


@jax.jit
def kernel(points):
    return _voxelize(points, _GRID, _VOXEL_SIZE,
                     (_PC_RANGE[0], _PC_RANGE[1], _PC_RANGE[2]),
                     _MAX_VOX, _MAX_PTS)
